# baseline (speedup 1.0000x reference)
import jax
import jax.numpy as jnp
import numpy as np
from jax import lax
from jax.experimental import pallas as pl
from jax.experimental.pallas import tpu as pltpu
from jax.experimental.pallas import tpu_sc as plsc

_TOKENS = 32768
_D = 768
_E = 64
_TC_TILE = 4096
_NW = 32
_TPW = _TOKENS // _NW
_CH = 512
_NCH = _TPW // _CH
_NGRP = _CH // 16

_SIGN_LOW = np.int32(0x7FFFFFFF)
_IDX_MASK = np.int32(63)
_HI_MASK = np.int32(-64)
_I32_MIN = np.int32(-(2**31))


def _keys_body(x_ref, w_ref, b_ref, o_ref):
    w = w_ref[...]
    bias = b_ref[...]
    for j in range(_TC_TILE // _TPW):
        x = x_ref[pl.ds(j * _TPW, _TPW), :]
        lt = lax.dot_general(
            w, x, (((1,), (1,)), ((), ())),
            preferred_element_type=jnp.float32) + bias
        bits = lax.bitcast_convert_type(lt, jnp.int32)
        key = bits ^ ((bits >> 31) & _SIGN_LOW)
        eidx = lax.broadcasted_iota(jnp.int32, lt.shape, 0)
        key = (key & _HI_MASK) | (_IDX_MASK - eidx)
        for ci in range(_NCH):
            o_ref[j, ci] = key[:, ci * _CH:(ci + 1) * _CH]


def _route_body(keys_hbm, out_hbm, in_buf, out_buf, stash):
    cid = lax.axis_index("c")
    sid = lax.axis_index("s")
    wid = sid * 2 + cid
    lane = lax.iota(jnp.int32, 16)
    zero16 = jnp.zeros((16,), jnp.float32)

    for c4 in range(_E // 16):
        def zbody(r, _, c4=c4):
            out_buf[r, pl.ds(c4 * 16, 16)] = zero16
            return 0
        lax.fori_loop(0, _CH, zbody, 0)

    for c in range(_NCH):
        pltpu.sync_copy(keys_hbm.at[wid, c], in_buf)
        if c > 0:
            def rzbody(g, _):
                row = lane + g * 16
                plsc.store_scatter(out_buf, [row, stash[2 * g]], zero16)
                plsc.store_scatter(out_buf, [row, stash[2 * g + 1]], zero16)
                return 0
            lax.fori_loop(0, _NGRP, rzbody, 0)

        def gbody(g, _):
            tok = g * 16
            m1 = in_buf[0, pl.ds(tok, 16)]
            m2 = jnp.full((16,), _I32_MIN)

            def ebody(e, carry):
                m1, m2 = carry
                v = in_buf[e, pl.ds(tok, 16)]
                t = jnp.minimum(v, m1)
                return jnp.maximum(v, m1), jnp.maximum(m2, t)

            m1, m2 = lax.fori_loop(1, _E, ebody, (m1, m2))
            e1 = _IDX_MASK - (m1 & _IDX_MASK)
            e2 = _IDX_MASK - (m2 & _IDX_MASK)
            v1 = plsc.bitcast(m1 ^ ((m1 >> 31) & _SIGN_LOW), jnp.float32)
            v2 = plsc.bitcast(m2 ^ ((m2 >> 31) & _SIGN_LOW), jnp.float32)
            s = jnp.exp(v2 - v1)
            w2 = s / (1.0 + s)
            w1 = 1.0 - w2
            row = lane + g * 16
            plsc.store_scatter(out_buf, [row, e1], w1)
            plsc.store_scatter(out_buf, [row, e2], w2)
            stash[2 * g] = e1
            stash[2 * g + 1] = e2
            return 0

        lax.fori_loop(0, _NGRP, gbody, 0)
        tok0 = wid * _TPW + c * _CH
        pltpu.sync_copy(out_buf, out_hbm.at[pl.ds(tok0, _CH)])


def kernel(payload_tensor, W, b):
    b2 = b.reshape(_E, 1)
    keys = pl.pallas_call(
        _keys_body,
        grid=(_TOKENS // _TC_TILE,),
        in_specs=[
            pl.BlockSpec((_TC_TILE, _D), lambda i: (i, 0)),
            pl.BlockSpec((_E, _D), lambda i: (0, 0)),
            pl.BlockSpec((_E, 1), lambda i: (0, 0)),
        ],
        out_specs=pl.BlockSpec(
            (_TC_TILE // _TPW, _NCH, _E, _CH), lambda i: (i, 0, 0, 0)),
        out_shape=jax.ShapeDtypeStruct((_NW, _NCH, _E, _CH), jnp.int32),
    )(payload_tensor, W, b2)

    route = pl.kernel(
        _route_body,
        out_type=jax.ShapeDtypeStruct((_TOKENS, _E), jnp.float32),
        mesh=plsc.VectorSubcoreMesh(core_axis_name="c", subcore_axis_name="s"),
        compiler_params=pltpu.CompilerParams(needs_layout_passes=False),
        scratch_types=[
            pltpu.VMEM((_E, _CH), jnp.int32),
            pltpu.VMEM((_CH, _E), jnp.float32),
            pltpu.VMEM((2 * _NGRP, 16), jnp.int32),
        ],
    )
    return route(keys)


# speedup vs baseline: 1.1880x; 1.1880x over previous
import jax
import jax.numpy as jnp
import numpy as np
from jax import lax
from jax.experimental import pallas as pl
from jax.experimental.pallas import tpu as pltpu
from jax.experimental.pallas import tpu_sc as plsc

_TOKENS = 32768
_D = 768
_E = 64
_TC_TILE = 4096
_NW = 32
_TPW = _TOKENS // _NW
_CH = 512
_NCH = _TPW // _CH
_NGRP = _CH // 16

_NEG_INF = np.float32(-np.inf)


def _logits_body(x_ref, w_ref, b_ref, o_ref):
    w = w_ref[...]
    bias = b_ref[...]
    for j in range(_TC_TILE // _TPW):
        x = x_ref[pl.ds(j * _TPW, _TPW), :]
        lt = lax.dot_general(
            w, x, (((1,), (1,)), ((), ())),
            preferred_element_type=jnp.float32) + bias
        for ci in range(_NCH):
            o_ref[j, ci] = lt[:, ci * _CH:(ci + 1) * _CH]


def _argcmp_merge(mv, mi, cv, ci):
    take = (cv > mv) | ((cv == mv) & (ci < mi))
    return jnp.where(take, cv, mv), jnp.where(take, ci, mi)


def _route_body(lg_hbm, out_hbm, in_buf, out_buf, stash):
    cid = lax.axis_index("c")
    sid = lax.axis_index("s")
    wid = sid * 2 + cid
    lane = lax.iota(jnp.int32, 16)
    zero16 = jnp.zeros((16,), jnp.float32)

    def zbody(r, _):
        for c4 in range(_E // 16):
            out_buf[r, pl.ds(c4 * 16, 16)] = zero16
        return 0
    lax.fori_loop(0, _CH, zbody, 0)

    for c in range(_NCH):
        pltpu.sync_copy(lg_hbm.at[wid, c], in_buf)
        if c > 0:
            def rzbody(g, _):
                row = lane + g * 16
                plsc.store_scatter(out_buf, [row, stash[2 * g]], zero16)
                plsc.store_scatter(out_buf, [row, stash[2 * g + 1]], zero16)
                return 0
            lax.fori_loop(0, _NGRP, rzbody, 0)

        def gbody(g, _):
            tok = g * 16
            m1a = in_buf[0, pl.ds(tok, 16)]
            m1b = in_buf[1, pl.ds(tok, 16)]
            i1a = jnp.zeros((16,), jnp.int32)
            i1b = jnp.ones((16,), jnp.int32)
            m2a = jnp.full((16,), _NEG_INF)
            m2b = jnp.full((16,), _NEG_INF)
            i2a = jnp.zeros((16,), jnp.int32)
            i2b = jnp.zeros((16,), jnp.int32)
            for e in range(2, _E, 2):
                va = in_buf[e, pl.ds(tok, 16)]
                vb = in_buf[e + 1, pl.ds(tok, 16)]
                ea = jnp.full((16,), np.int32(e))
                eb = jnp.full((16,), np.int32(e + 1))
                ca1 = va > m1a
                cb1 = vb > m1b
                ca2 = va > m2a
                cb2 = vb > m2b
                m2a = jnp.where(ca1, m1a, jnp.where(ca2, va, m2a))
                i2a = jnp.where(ca1, i1a, jnp.where(ca2, ea, i2a))
                m2b = jnp.where(cb1, m1b, jnp.where(cb2, vb, m2b))
                i2b = jnp.where(cb1, i1b, jnp.where(cb2, eb, i2b))
                m1a = jnp.where(ca1, va, m1a)
                i1a = jnp.where(ca1, ea, i1a)
                m1b = jnp.where(cb1, vb, m1b)
                i1b = jnp.where(cb1, eb, i1b)
            takeb = (m1b > m1a) | ((m1b == m1a) & (i1b < i1a))
            v1 = jnp.where(takeb, m1b, m1a)
            e1 = jnp.where(takeb, i1b, i1a)
            lv = jnp.where(takeb, m1a, m1b)
            li = jnp.where(takeb, i1a, i1b)
            v2, e2 = _argcmp_merge(lv, li, m2a, i2a)
            v2, e2 = _argcmp_merge(v2, e2, m2b, i2b)
            s = jnp.exp(v2 - v1)
            w2 = s / (1.0 + s)
            w1 = 1.0 - w2
            row = lane + g * 16
            plsc.store_scatter(out_buf, [row, e1], w1)
            plsc.store_scatter(out_buf, [row, e2], w2)
            stash[2 * g] = e1
            stash[2 * g + 1] = e2
            return 0

        lax.fori_loop(0, _NGRP, gbody, 0)
        tok0 = wid * _TPW + c * _CH
        pltpu.sync_copy(out_buf, out_hbm.at[pl.ds(tok0, _CH)])


def kernel(payload_tensor, W, b):
    b2 = b.reshape(_E, 1)
    logits = pl.pallas_call(
        _logits_body,
        grid=(_TOKENS // _TC_TILE,),
        in_specs=[
            pl.BlockSpec((_TC_TILE, _D), lambda i: (i, 0)),
            pl.BlockSpec((_E, _D), lambda i: (0, 0)),
            pl.BlockSpec((_E, 1), lambda i: (0, 0)),
        ],
        out_specs=pl.BlockSpec(
            (_TC_TILE // _TPW, _NCH, _E, _CH), lambda i: (i, 0, 0, 0)),
        out_shape=jax.ShapeDtypeStruct((_NW, _NCH, _E, _CH), jnp.float32),
    )(payload_tensor, W, b2)

    route = pl.kernel(
        _route_body,
        out_type=jax.ShapeDtypeStruct((_TOKENS, _E), jnp.float32),
        mesh=plsc.VectorSubcoreMesh(core_axis_name="c", subcore_axis_name="s"),
        compiler_params=pltpu.CompilerParams(needs_layout_passes=False),
        scratch_types=[
            pltpu.VMEM((_E, _CH), jnp.float32),
            pltpu.VMEM((_CH, _E), jnp.float32),
            pltpu.VMEM((2 * _NGRP, 16), jnp.int32),
        ],
    )
    return route(logits)
